# Initial kernel scaffold; baseline (speedup 1.0000x reference)
import jax
import jax.numpy as jnp
from jax.experimental import pallas as pl


def kernel(h, edge_index, coord, edge_attr, We1, be1, We2, be2, Wn1, bn1, Wn2, bn2, Wc1, bc1, Wc2):
    raise NotImplementedError("write your pallas kernel here")


# speedup vs baseline: 3.3169x; 3.3169x over previous
import jax
import jax.numpy as jnp
from jax import lax
from jax._src.pallas.core import CoreMemorySpace as _CoreMemorySpace
from jax.experimental import pallas as pl
from jax.experimental.pallas import tpu as pltpu
from jax.experimental.pallas import tpu_sc as plsc

N = 10000
E = 320000
D = 128
H = 128
DE = 4
NC = 2
NS = 16
NW = NC * NS
EPW = E // NW
CH = 80
NCH = EPW // CH
NNCH = N // CH
G = CH // 16

BN = 1000
BE = 512


def _silu(x):
    return x * jax.nn.sigmoid(x)


def _pre_body(h_ref, wa_ref, wb_ref, be1_ref, ta_ref, tb_ref):
    h = h_ref[...]
    ta_ref[...] = jnp.dot(h, wa_ref[...], preferred_element_type=jnp.float32) + be1_ref[...]
    tb_ref[...] = jnp.dot(h, wb_ref[...], preferred_element_type=jnp.float32)


def _pre_call(h, wa_t, wb_t, be1r):
    grid = (N // BN,)
    return pl.pallas_call(
        _pre_body,
        grid=grid,
        in_specs=[
            pl.BlockSpec((BN, D), lambda i: (i, 0)),
            pl.BlockSpec((D, H), lambda i: (0, 0)),
            pl.BlockSpec((D, H), lambda i: (0, 0)),
            pl.BlockSpec((1, H), lambda i: (0, 0)),
        ],
        out_specs=[
            pl.BlockSpec((BN, H), lambda i: (i, 0)),
            pl.BlockSpec((BN, H), lambda i: (i, 0)),
        ],
        out_shape=[
            jax.ShapeDtypeStruct((N, H), jnp.float32),
            jax.ShapeDtypeStruct((N, H), jnp.float32),
        ],
    )(h, wa_t, wb_t, be1r)


def _g_chunk_compute(idxr, idxc, cdbuf, ct, iota, c3):
    for g in range(G):
        ir = idxr[pl.ds(g * 16, 16)]
        ic = idxc[pl.ds(g * 16, 16)]
        rr = g * 16 + iota
        r2 = jnp.zeros((16,), jnp.float32)
        for cc in range(3):
            ccv = jnp.full((16,), cc, jnp.int32)
            xr = plsc.load_gather(ct, [ir * 4 + ccv])
            xc = plsc.load_gather(ct, [ic * 4 + ccv])
            dv = xr - xc
            plsc.store_scatter(cdbuf, [rr, ccv], dv)
            r2 = r2 + dv * dv
        plsc.store_scatter(cdbuf, [rr, c3], r2)


def _gather_body(ta_hbm, tb_hbm, row_hbm, col_hbm, cp4_hbm,
                 xa_hbm, xb_hbm, cd_hbm,
                 idxr0, idxc0, bufa0, bufb0, cdbuf0,
                 idxr1, idxc1, bufa1, bufb1, cdbuf1, ct,
                 semi0, semi1, semg0, semg1, semw0, semw1):
    c = lax.axis_index("c")
    s = lax.axis_index("s")
    base = (c * NS + s) * EPW

    pltpu.sync_copy(cp4_hbm, ct)
    iota = lax.iota(jnp.int32, 16)
    c3 = jnp.full((16,), 3, jnp.int32)

    sets = (
        (idxr0, idxc0, bufa0, bufb0, cdbuf0, semi0, semg0, semw0),
        (idxr1, idxc1, bufa1, bufb1, cdbuf1, semi1, semg1, semw1),
    )

    def fire_idx(j, b):
        idxr, idxc = sets[b][0], sets[b][1]
        semi = sets[b][5]
        off = base + j * CH
        pltpu.async_copy(row_hbm.at[pl.ds(off, CH)], idxr, semi)
        pltpu.async_copy(col_hbm.at[pl.ds(off, CH)], idxc, semi)

    def wait_idx(b):
        idxr, idxc = sets[b][0], sets[b][1]
        semi = sets[b][5]
        pltpu.make_async_copy(row_hbm.at[pl.ds(0, CH)], idxr, semi).wait()
        pltpu.make_async_copy(col_hbm.at[pl.ds(0, CH)], idxc, semi).wait()

    def fire_gather(b):
        idxr, idxc, bufa, bufb = sets[b][:4]
        semg = sets[b][6]
        pltpu.async_copy(ta_hbm.at[idxr], bufa, semg)
        pltpu.async_copy(tb_hbm.at[idxc], bufb, semg)

    def wait_gather(b):
        idxr, idxc, bufa, bufb = sets[b][:4]
        semg = sets[b][6]
        pltpu.make_async_copy(ta_hbm.at[idxr], bufa, semg).wait()
        pltpu.make_async_copy(tb_hbm.at[idxc], bufb, semg).wait()

    def fire_writes(j, b):
        bufa, bufb, cdbuf = sets[b][2], sets[b][3], sets[b][4]
        semw = sets[b][7]
        off = base + j * CH
        pltpu.async_copy(bufa, xa_hbm.at[pl.ds(off, CH)], semw)
        pltpu.async_copy(bufb, xb_hbm.at[pl.ds(off, CH)], semw)
        pltpu.async_copy(cdbuf, cd_hbm.at[pl.ds(off, CH)], semw)

    def wait_writes(b):
        bufa, bufb, cdbuf = sets[b][2], sets[b][3], sets[b][4]
        semw = sets[b][7]
        pltpu.make_async_copy(bufa, xa_hbm.at[pl.ds(0, CH)], semw).wait()
        pltpu.make_async_copy(bufb, xb_hbm.at[pl.ds(0, CH)], semw).wait()
        pltpu.make_async_copy(cdbuf, cd_hbm.at[pl.ds(0, CH)], semw).wait()

    def step(j, b, first, last):
        wait_gather(b)
        if not last:
            wait_idx(1 - b)
            if not first:
                wait_writes(1 - b)
            fire_gather(1 - b)

        idxr, idxc, cdbuf = sets[b][0], sets[b][1], sets[b][4]
        _g_chunk_compute(idxr, idxc, cdbuf, ct, iota, c3)
        if not last:
            @pl.when(j + 2 < NCH)
            def _():
                fire_idx(j + 2, b)

        fire_writes(j, b)

    fire_idx(0, 0)
    wait_idx(0)
    fire_gather(0)
    fire_idx(1, 1)

    def pair(t, carry):
        j0 = 2 * t
        step(j0, 0, False, False)
        step(j0 + 1, 1, False, False)
        return carry

    step(0, 0, True, False)
    step(1, 1, False, False)
    lax.fori_loop(1, NCH // 2, pair, 0)
    step(NCH - 1, 0, False, True)
    wait_writes(0)
    wait_writes(1)


def _gather_call(ta, tb, row, col, cp4):
    mesh = plsc.VectorSubcoreMesh(core_axis_name="c", subcore_axis_name="s",
                                  num_cores=NC, num_subcores=NS)
    f = pl.kernel(
        _gather_body,
        out_type=(
            jax.ShapeDtypeStruct((E, H), jnp.float32),
            jax.ShapeDtypeStruct((E, H), jnp.float32),
            jax.ShapeDtypeStruct((E, 16), jnp.float32),
        ),
        mesh=mesh,
        compiler_params=pltpu.CompilerParams(needs_layout_passes=False),
        scratch_types=(
            pltpu.VMEM((CH,), jnp.int32),
            pltpu.VMEM((CH,), jnp.int32),
            pltpu.VMEM((CH, H), jnp.float32),
            pltpu.VMEM((CH, H), jnp.float32),
            pltpu.VMEM((CH, 16), jnp.float32),
            pltpu.VMEM((CH,), jnp.int32),
            pltpu.VMEM((CH,), jnp.int32),
            pltpu.VMEM((CH, H), jnp.float32),
            pltpu.VMEM((CH, H), jnp.float32),
            pltpu.VMEM((CH, 16), jnp.float32),
            pltpu.VMEM((N * 4,), jnp.float32),
            pltpu.SemaphoreType.DMA,
            pltpu.SemaphoreType.DMA,
            pltpu.SemaphoreType.DMA,
            pltpu.SemaphoreType.DMA,
            pltpu.SemaphoreType.DMA,
            pltpu.SemaphoreType.DMA,
        ),
    )
    return f(ta, tb, row, col, cp4)


def _edge_body(xa_ref, xb_ref, cd_ref, ea_ref, wr_ref, wea_ref, we2_ref,
               be2_ref, wc1_ref, bc1_ref, wc2_ref, eftr_ref):
    cd = cd_ref[...]
    lane = lax.broadcasted_iota(jnp.int32, (BE, 16), 1)
    r = jnp.sum(jnp.where(lane == 3, cd, 0.0), axis=1, keepdims=True)
    pre = xa_ref[...] + xb_ref[...] + r * wr_ref[...] + jnp.dot(
        ea_ref[...], wea_ref[...], preferred_element_type=jnp.float32)
    m = _silu(pre)
    ef = _silu(jnp.dot(m, we2_ref[...], preferred_element_type=jnp.float32)
               + be2_ref[...])
    ch = _silu(jnp.dot(ef, wc1_ref[...], preferred_element_type=jnp.float32)
               + bc1_ref[...])
    phi = jnp.sum(ch * wc2_ref[...], axis=1, keepdims=True)
    tr16 = jnp.where(lane == 3, 1.0, jnp.where(lane < 3, cd * phi, 0.0))
    eftr_ref[...] = jnp.concatenate(
        [ef, tr16, jnp.zeros((BE, 112), jnp.float32)], axis=1)


def _edge_call(xa, xb, cd, ea, wr, wea_t, we2_t, be2r, wc1_t, bc1r, wc2r):
    grid = (E // BE,)
    full = lambda shape: pl.BlockSpec(shape, lambda i: (0, 0))
    return pl.pallas_call(
        _edge_body,
        grid=grid,
        in_specs=[
            pl.BlockSpec((BE, H), lambda i: (i, 0)),
            pl.BlockSpec((BE, H), lambda i: (i, 0)),
            pl.BlockSpec((BE, 16), lambda i: (i, 0)),
            pl.BlockSpec((BE, DE), lambda i: (i, 0)),
            full((1, H)),
            full((DE, H)),
            full((H, H)),
            full((1, H)),
            full((H, H)),
            full((1, H)),
            full((1, H)),
        ],
        out_specs=[
            pl.BlockSpec((BE, 2 * H), lambda i: (i, 0)),
        ],
        out_shape=[
            jax.ShapeDtypeStruct((E, 2 * H), jnp.float32),
        ],
    )(xa, xb, cd, ea, wr, wea_t, we2_t, be2r, wc1_t, bc1r, wc2r)


NPT = 320
CH2 = 2000
NG2 = CH2 // 16
SCAP = 2048
LW = E + SCAP


def _scan_body(row_hbm, plist_hbm, cnt_hbm,
               rows0, rows1, est, cntbuf, sem0, sem1):
    c = lax.axis_index("c")
    s = lax.axis_index("s")
    w = c * NS + s
    lo = w * NPT
    iota = lax.iota(jnp.int32, 16)
    base = w * LW

    def scan_rows(rows, carry):
        def group_body(g, carry):
            cnt, nf, idsp = carry
            v = rows[pl.ds(g * 16, 16)]
            m = (v >= lo) & (v < lo + NPT)
            plsc.store_compressed(est.at[pl.ds(cnt, 16)],
                                  idsp + (v - lo), mask=m)
            cnt = cnt + jnp.sum(m.astype(jnp.int32))

            @pl.when(cnt >= SCAP)
            def _():
                pltpu.sync_copy(est.at[pl.ds(0, SCAP)],
                                plist_hbm.at[pl.ds(base + nf * SCAP, SCAP)])
                est[pl.ds(0, 16)] = est[pl.ds(SCAP, 16)]

            hit = (cnt >= SCAP).astype(jnp.int32)
            return (cnt - hit * SCAP, nf + hit, idsp + 16 * 512)

        return lax.fori_loop(0, NG2, group_body, carry)

    nd = E // CH2 // 2

    def chunk_pair(t, carry):
        pltpu.make_async_copy(row_hbm.at[pl.ds(0, CH2)], rows0, sem0).wait()
        pltpu.async_copy(row_hbm.at[pl.ds((2 * t + 1) * CH2, CH2)],
                         rows1, sem1)
        carry = scan_rows(rows0, carry)

        @pl.when(t + 1 < nd)
        def _():
            pltpu.async_copy(row_hbm.at[pl.ds((2 * t + 2) * CH2, CH2)],
                             rows0, sem0)

        pltpu.make_async_copy(row_hbm.at[pl.ds(0, CH2)], rows1, sem1).wait()
        carry = scan_rows(rows1, carry)
        return carry

    pltpu.async_copy(row_hbm.at[pl.ds(0, CH2)], rows0, sem0)
    cnt, nf, _ = lax.fori_loop(
        0, nd, chunk_pair,
        (jnp.full((), 0, jnp.int32), jnp.full((), 0, jnp.int32),
         iota * 512))

    pltpu.sync_copy(est.at[pl.ds(0, SCAP)],
                    plist_hbm.at[pl.ds(base + nf * SCAP, SCAP)])
    cntbuf[pl.ds(0, 16)] = jnp.broadcast_to(nf * SCAP + cnt, (16,))
    pltpu.sync_copy(cntbuf, cnt_hbm.at[pl.ds(w * 16, 16)])


def _scan_call(row):
    mesh = plsc.VectorSubcoreMesh(core_axis_name="c", subcore_axis_name="s",
                                  num_cores=NC, num_subcores=NS)
    f = pl.kernel(
        _scan_body,
        out_type=(
            jax.ShapeDtypeStruct((NW * LW,), jnp.int32),
            jax.ShapeDtypeStruct((NW * 16,), jnp.int32),
        ),
        mesh=mesh,
        compiler_params=pltpu.CompilerParams(needs_layout_passes=False),
        scratch_types=(
            pltpu.VMEM((CH2,), jnp.int32),
            pltpu.VMEM((CH2,), jnp.int32),
            pltpu.VMEM((SCAP + 32,), jnp.int32),
            pltpu.VMEM((16,), jnp.int32),
            pltpu.SemaphoreType.DMA,
            pltpu.SemaphoreType.DMA,
        ),
    )
    return f(row)


def _accum_body(eftr_hbm, plist_hbm, cnt_hbm, zf_hbm, zc_hbm,
                ag_hbm, ac_hbm,
                land0, land1, seb0, srb0, seb1, srb1, gbuf0, gbuf1,
                acc16, accf, cbuf, semi0, semi1, semg0, semg1):
    c = lax.axis_index("c")
    s = lax.axis_index("s")
    w = c * NS + s
    iota = lax.iota(jnp.int32, 16)
    base = w * LW

    pltpu.sync_copy(zf_hbm, accf)
    pltpu.sync_copy(zc_hbm, acc16)
    pltpu.sync_copy(cnt_hbm.at[pl.ds(w * 16, 16)], cbuf)
    cnt = jnp.sum(jnp.where(iota == 0, cbuf[pl.ds(0, 16)], 0))
    nb = (cnt + CH - 1) // CH

    sets = ((land0, seb0, srb0, gbuf0, semi0, semg0),
            (land1, seb1, srb1, gbuf1, semi1, semg1))

    def fire_idx(j, b):
        land, semi = sets[b][0], sets[b][4]
        pltpu.async_copy(plist_hbm.at[pl.ds(base + j * CH, CH)], land, semi)

    def wait_idx(b):
        land, semi = sets[b][0], sets[b][4]
        pltpu.make_async_copy(plist_hbm.at[pl.ds(0, CH)], land, semi).wait()

    def snapshot(j, b):
        land, seb, srb = sets[b][0], sets[b][1], sets[b][2]
        for k in range(CH // 16):
            pv = land[pl.ds(k * 16, 16)]
            sel = (j * CH + k * 16 + iota) < cnt
            eb = pv // 512
            rb = pv - eb * 512
            seb[pl.ds(k * 16, 16)] = jnp.where(sel, eb, 0)
            srb[pl.ds(k * 16, 16)] = jnp.where(
                sel, rb, jnp.full((16,), NPT, jnp.int32))

    def fire_gather(b):
        seb, gbuf, semg = sets[b][1], sets[b][3], sets[b][5]
        pltpu.async_copy(eftr_hbm.at[seb], gbuf, semg)

    def wait_gather(b):
        seb, gbuf, semg = sets[b][1], sets[b][3], sets[b][5]
        pltpu.make_async_copy(eftr_hbm.at[seb], gbuf, semg).wait()

    def accum(b):
        srb, gbuf = sets[b][2], sets[b][3]
        for j in range(CH):
            k, l = divmod(j, 16)
            rv = srb[pl.ds(k * 16, 16)]
            rj = rv.at[jnp.full((16,), l, jnp.int32)].get(
                mode="promise_in_bounds")
            mask = rj < NPT
            plsc.addupdate_scatter(acc16, [rj * 16 + iota],
                                   gbuf[j, pl.ds(H, 16)], mask=mask)
            rbase = rj * H + iota
            for g in range(H // 16):
                plsc.addupdate_scatter(accf, [rbase + g * 16],
                                       gbuf[j, pl.ds(g * 16, 16)], mask=mask)

    def step(j, b):
        wait_gather(b)

        @pl.when(j + 1 < nb)
        def _():
            wait_idx(1 - b)
            snapshot(j + 1, 1 - b)
            fire_gather(1 - b)

            @pl.when(j + 2 < nb)
            def _():
                fire_idx(j + 2, b)

        accum(b)

    @pl.when(nb > 0)
    def _():
        fire_idx(0, 0)
        wait_idx(0)
        snapshot(jnp.full((), 0, jnp.int32), 0)
        fire_gather(0)

        @pl.when(nb > 1)
        def _():
            fire_idx(1, 1)

    def pair(t, carry):
        j0 = 2 * t

        @pl.when(j0 < nb)
        def _():
            step(j0, 0)

        @pl.when(j0 + 1 < nb)
        def _():
            step(j0 + 1, 1)

        return carry

    lax.fori_loop(0, (nb + 1) // 2, pair, jnp.full((), 0, jnp.int32))

    pltpu.sync_copy(accf, ag_hbm.at[pl.ds(w * NPT * H, NPT * H)])
    pltpu.sync_copy(acc16, ac_hbm.at[pl.ds(w * NPT * 16, NPT * 16)])


def _accum_call(eftr, plist, pcnt, zf, zc):
    mesh = plsc.VectorSubcoreMesh(core_axis_name="c", subcore_axis_name="s",
                                  num_cores=NC, num_subcores=NS)
    f = pl.kernel(
        _accum_body,
        out_type=(
            jax.ShapeDtypeStruct((NW * NPT * H,), jnp.float32),
            jax.ShapeDtypeStruct((NW * NPT * 16,), jnp.float32),
        ),
        mesh=mesh,
        compiler_params=pltpu.CompilerParams(needs_layout_passes=False),
        scratch_types=(
            pltpu.VMEM((CH,), jnp.int32),
            pltpu.VMEM((CH,), jnp.int32),
            pltpu.VMEM((CH,), jnp.int32),
            pltpu.VMEM((CH,), jnp.int32),
            pltpu.VMEM((CH,), jnp.int32),
            pltpu.VMEM((CH,), jnp.int32),
            pltpu.VMEM((CH, 2 * H), jnp.float32),
            pltpu.VMEM((CH, 2 * H), jnp.float32),
            pltpu.VMEM((NPT * 16,), jnp.float32),
            pltpu.VMEM((NPT * H,), jnp.float32),
            pltpu.VMEM((16,), jnp.int32),
            pltpu.SemaphoreType.DMA,
            pltpu.SemaphoreType.DMA,
            pltpu.SemaphoreType.DMA,
            pltpu.SemaphoreType.DMA,
        ),
    )
    return f(eftr, plist, pcnt, zf, zc)


def _node_body(h_ref, ag_ref, ac_ref, cp_ref,
               wn1a_ref, wn1b_ref, bn1_ref, wn2_ref, bn2_ref,
               ho_ref, co_ref):
    h = h_ref[...]
    aggh = ag_ref[...]
    aggc = ac_ref[...]
    nm = _silu(jnp.dot(h, wn1a_ref[...], preferred_element_type=jnp.float32)
               + jnp.dot(aggh, wn1b_ref[...], preferred_element_type=jnp.float32)
               + bn1_ref[...])
    ho_ref[...] = h + jnp.dot(nm, wn2_ref[...], preferred_element_type=jnp.float32) + bn2_ref[...]
    lane = lax.broadcasted_iota(jnp.int32, (BN, 16), 1)
    cnt = jnp.sum(jnp.where(lane == 3, aggc, 0.0), axis=1, keepdims=True)
    co_ref[...] = cp_ref[...] + aggc[:, :4] / jnp.maximum(cnt, 1.0)


def _node_call(h, ag, ac, cp4, wn1a_t, wn1b_t, bn1r, wn2_t, bn2r):
    grid = (N // BN,)
    return pl.pallas_call(
        _node_body,
        grid=grid,
        in_specs=[
            pl.BlockSpec((BN, D), lambda i: (i, 0)),
            pl.BlockSpec((BN, H), lambda i: (i, 0)),
            pl.BlockSpec((BN, 16), lambda i: (i, 0)),
            pl.BlockSpec((BN, 4), lambda i: (i, 0)),
            pl.BlockSpec((D, H), lambda i: (0, 0)),
            pl.BlockSpec((H, H), lambda i: (0, 0)),
            pl.BlockSpec((1, H), lambda i: (0, 0)),
            pl.BlockSpec((H, D), lambda i: (0, 0)),
            pl.BlockSpec((1, D), lambda i: (0, 0)),
        ],
        out_specs=[
            pl.BlockSpec((BN, D), lambda i: (i, 0)),
            pl.BlockSpec((BN, 4), lambda i: (i, 0)),
        ],
        out_shape=[
            jax.ShapeDtypeStruct((N, D), jnp.float32),
            jax.ShapeDtypeStruct((N, 4), jnp.float32),
        ],
    )(h, ag, ac, cp4, wn1a_t, wn1b_t, bn1r, wn2_t, bn2r)


def kernel(h, edge_index, coord, edge_attr, We1, be1, We2, be2,
           Wn1, bn1, Wn2, bn2, Wc1, bc1, Wc2):
    row = edge_index[0].astype(jnp.int32)
    col = edge_index[1].astype(jnp.int32)
    cp4 = jnp.zeros((N, 4), jnp.float32).at[:, :3].set(coord)

    wa_t = We1[:, :D].T
    wb_t = We1[:, D:2 * D].T
    wr = We1[:, 2 * D].reshape(1, H)
    wea_t = We1[:, 2 * D + 1:].T
    we2_t = We2.T
    wc1_t = Wc1.T
    wc2r = Wc2.reshape(1, H)
    wn1a_t = Wn1[:, :D].T
    wn1b_t = Wn1[:, D:].T
    wn2_t = Wn2.T

    ta, tb = _pre_call(h, wa_t, wb_t, be1.reshape(1, H))
    xa, xb, cd = _gather_call(ta, tb, row, col, cp4.reshape(-1))
    plist, pcnt = _scan_call(row)
    (eftr,) = _edge_call(xa, xb, cd, edge_attr, wr, wea_t, we2_t,
                      be2.reshape(1, H), wc1_t, bc1.reshape(1, H), wc2r)
    zf = jnp.zeros((NPT * H,), jnp.float32)
    zc = jnp.zeros((NPT * 16,), jnp.float32)
    ag, ac = _accum_call(eftr, plist, pcnt, zf, zc)
    ag = ag.reshape(NW * NPT, H)
    ac = ac.reshape(NW * NPT, 16)
    ho, co = _node_call(h, ag, ac, cp4, wn1a_t, wn1b_t,
                        bn1.reshape(1, H), wn2_t, bn2.reshape(1, D))
    return (ho, co[:, :3], edge_attr)


# speedup vs baseline: 3.4892x; 1.0520x over previous
import jax
import jax.numpy as jnp
from jax import lax
from jax._src.pallas.core import CoreMemorySpace as _CoreMemorySpace
from jax.experimental import pallas as pl
from jax.experimental.pallas import tpu as pltpu
from jax.experimental.pallas import tpu_sc as plsc

N = 10000
E = 320000
D = 128
H = 128
DE = 4
NC = 2
NS = 16
NW = NC * NS
EPW = E // NW
CH = 80
NCH = EPW // CH
NNCH = N // CH
G = CH // 16

BN = 1000
BE = 512


def _silu(x):
    return x * jax.nn.sigmoid(x)


def _pre_body(h_ref, wa_ref, wb_ref, be1_ref, ta_ref, tb_ref):
    h = h_ref[...]
    ta_ref[...] = jnp.dot(h, wa_ref[...], preferred_element_type=jnp.float32) + be1_ref[...]
    tb_ref[...] = jnp.dot(h, wb_ref[...], preferred_element_type=jnp.float32)


def _pre_call(h, wa_t, wb_t, be1r):
    grid = (N // BN,)
    return pl.pallas_call(
        _pre_body,
        grid=grid,
        in_specs=[
            pl.BlockSpec((BN, D), lambda i: (i, 0)),
            pl.BlockSpec((D, H), lambda i: (0, 0)),
            pl.BlockSpec((D, H), lambda i: (0, 0)),
            pl.BlockSpec((1, H), lambda i: (0, 0)),
        ],
        out_specs=[
            pl.BlockSpec((BN, H), lambda i: (i, 0)),
            pl.BlockSpec((BN, H), lambda i: (i, 0)),
        ],
        out_shape=[
            jax.ShapeDtypeStruct((N, H), jnp.float32),
            jax.ShapeDtypeStruct((N, H), jnp.float32),
        ],
    )(h, wa_t, wb_t, be1r)


def _g_chunk_compute(idxr, idxc, cdbuf, ct, iota, c3):
    for g in range(G):
        ir = idxr[pl.ds(g * 16, 16)]
        ic = idxc[pl.ds(g * 16, 16)]
        rr = g * 16 + iota
        r2 = jnp.zeros((16,), jnp.float32)
        for cc in range(3):
            ccv = jnp.full((16,), cc, jnp.int32)
            xr = plsc.load_gather(ct, [ir * 4 + ccv])
            xc = plsc.load_gather(ct, [ic * 4 + ccv])
            dv = xr - xc
            plsc.store_scatter(cdbuf, [rr, ccv], dv)
            r2 = r2 + dv * dv
        plsc.store_scatter(cdbuf, [rr, c3], r2)


def _gather_body(ta_hbm, tb_hbm, row_hbm, col_hbm, cp4_hbm,
                 xa_hbm, xb_hbm, cd_hbm,
                 idxr0, idxc0, bufa0, bufb0, cdbuf0,
                 idxr1, idxc1, bufa1, bufb1, cdbuf1, ct,
                 semi0, semi1, semg0, semg1, semw0, semw1):
    c = lax.axis_index("c")
    s = lax.axis_index("s")
    base = (c * NS + s) * EPW

    pltpu.sync_copy(cp4_hbm, ct)
    iota = lax.iota(jnp.int32, 16)
    c3 = jnp.full((16,), 3, jnp.int32)

    sets = (
        (idxr0, idxc0, bufa0, bufb0, cdbuf0, semi0, semg0, semw0),
        (idxr1, idxc1, bufa1, bufb1, cdbuf1, semi1, semg1, semw1),
    )

    def fire_idx(j, b):
        idxr, idxc = sets[b][0], sets[b][1]
        semi = sets[b][5]
        off = base + j * CH
        pltpu.async_copy(row_hbm.at[pl.ds(off, CH)], idxr, semi)
        pltpu.async_copy(col_hbm.at[pl.ds(off, CH)], idxc, semi)

    def wait_idx(b):
        idxr, idxc = sets[b][0], sets[b][1]
        semi = sets[b][5]
        pltpu.make_async_copy(row_hbm.at[pl.ds(0, CH)], idxr, semi).wait()
        pltpu.make_async_copy(col_hbm.at[pl.ds(0, CH)], idxc, semi).wait()

    def fire_gather(b):
        idxr, idxc, bufa, bufb = sets[b][:4]
        semg = sets[b][6]
        pltpu.async_copy(ta_hbm.at[idxr], bufa, semg)
        pltpu.async_copy(tb_hbm.at[idxc], bufb, semg)

    def wait_gather(b):
        idxr, idxc, bufa, bufb = sets[b][:4]
        semg = sets[b][6]
        pltpu.make_async_copy(ta_hbm.at[idxr], bufa, semg).wait()
        pltpu.make_async_copy(tb_hbm.at[idxc], bufb, semg).wait()

    def fire_writes(j, b):
        bufa, bufb, cdbuf = sets[b][2], sets[b][3], sets[b][4]
        semw = sets[b][7]
        off = base + j * CH
        pltpu.async_copy(bufa, xa_hbm.at[pl.ds(off, CH)], semw)
        pltpu.async_copy(bufb, xb_hbm.at[pl.ds(off, CH)], semw)
        pltpu.async_copy(cdbuf, cd_hbm.at[pl.ds(off, CH)], semw)

    def wait_writes(b):
        bufa, bufb, cdbuf = sets[b][2], sets[b][3], sets[b][4]
        semw = sets[b][7]
        pltpu.make_async_copy(bufa, xa_hbm.at[pl.ds(0, CH)], semw).wait()
        pltpu.make_async_copy(bufb, xb_hbm.at[pl.ds(0, CH)], semw).wait()
        pltpu.make_async_copy(cdbuf, cd_hbm.at[pl.ds(0, CH)], semw).wait()

    def step(j, b, first, last):
        wait_gather(b)
        if not last:
            wait_idx(1 - b)
            if not first:
                wait_writes(1 - b)
            fire_gather(1 - b)

        idxr, idxc, cdbuf = sets[b][0], sets[b][1], sets[b][4]
        _g_chunk_compute(idxr, idxc, cdbuf, ct, iota, c3)
        if not last:
            @pl.when(j + 2 < NCH)
            def _():
                fire_idx(j + 2, b)

        fire_writes(j, b)

    fire_idx(0, 0)
    wait_idx(0)
    fire_gather(0)
    fire_idx(1, 1)

    def pair(t, carry):
        j0 = 2 * t
        step(j0, 0, False, False)
        step(j0 + 1, 1, False, False)
        return carry

    step(0, 0, True, False)
    step(1, 1, False, False)
    lax.fori_loop(1, NCH // 2, pair, 0)
    step(NCH - 1, 0, False, True)
    wait_writes(0)
    wait_writes(1)


def _gather_call(ta, tb, row, col, cp4):
    mesh = plsc.VectorSubcoreMesh(core_axis_name="c", subcore_axis_name="s",
                                  num_cores=NC, num_subcores=NS)
    f = pl.kernel(
        _gather_body,
        out_type=(
            jax.ShapeDtypeStruct((E, H), jnp.float32),
            jax.ShapeDtypeStruct((E, H), jnp.float32),
            jax.ShapeDtypeStruct((E, 16), jnp.float32),
        ),
        mesh=mesh,
        compiler_params=pltpu.CompilerParams(needs_layout_passes=False),
        scratch_types=(
            pltpu.VMEM((CH,), jnp.int32),
            pltpu.VMEM((CH,), jnp.int32),
            pltpu.VMEM((CH, H), jnp.float32),
            pltpu.VMEM((CH, H), jnp.float32),
            pltpu.VMEM((CH, 16), jnp.float32),
            pltpu.VMEM((CH,), jnp.int32),
            pltpu.VMEM((CH,), jnp.int32),
            pltpu.VMEM((CH, H), jnp.float32),
            pltpu.VMEM((CH, H), jnp.float32),
            pltpu.VMEM((CH, 16), jnp.float32),
            pltpu.VMEM((N * 4,), jnp.float32),
            pltpu.SemaphoreType.DMA,
            pltpu.SemaphoreType.DMA,
            pltpu.SemaphoreType.DMA,
            pltpu.SemaphoreType.DMA,
            pltpu.SemaphoreType.DMA,
            pltpu.SemaphoreType.DMA,
        ),
    )
    return f(ta, tb, row, col, cp4)


def _edge_body(xa_ref, xb_ref, cd_ref, ea_ref, wr_ref, wea_ref, we2_ref,
               be2_ref, wc1_ref, bc1_ref, wc2_ref, eftr_ref):
    cd = cd_ref[...]
    lane = lax.broadcasted_iota(jnp.int32, (BE, 16), 1)
    r = jnp.sum(jnp.where(lane == 3, cd, 0.0), axis=1, keepdims=True)
    pre = xa_ref[...] + xb_ref[...] + r * wr_ref[...] + jnp.dot(
        ea_ref[...], wea_ref[...], preferred_element_type=jnp.float32)
    m = _silu(pre)
    ef = _silu(jnp.dot(m, we2_ref[...], preferred_element_type=jnp.float32)
               + be2_ref[...])
    ch = _silu(jnp.dot(ef, wc1_ref[...], preferred_element_type=jnp.float32)
               + bc1_ref[...])
    phi = jnp.sum(ch * wc2_ref[...], axis=1, keepdims=True)
    tr16 = jnp.where(lane == 3, 1.0, jnp.where(lane < 3, cd * phi, 0.0))
    eftr_ref[...] = jnp.concatenate(
        [ef, tr16, jnp.zeros((BE, 112), jnp.float32)], axis=1)


def _edge_call(xa, xb, cd, ea, wr, wea_t, we2_t, be2r, wc1_t, bc1r, wc2r):
    grid = (E // BE,)
    full = lambda shape: pl.BlockSpec(shape, lambda i: (0, 0))
    return pl.pallas_call(
        _edge_body,
        grid=grid,
        in_specs=[
            pl.BlockSpec((BE, H), lambda i: (i, 0)),
            pl.BlockSpec((BE, H), lambda i: (i, 0)),
            pl.BlockSpec((BE, 16), lambda i: (i, 0)),
            pl.BlockSpec((BE, DE), lambda i: (i, 0)),
            full((1, H)),
            full((DE, H)),
            full((H, H)),
            full((1, H)),
            full((H, H)),
            full((1, H)),
            full((1, H)),
        ],
        out_specs=[
            pl.BlockSpec((BE, 2 * H), lambda i: (i, 0)),
        ],
        out_shape=[
            jax.ShapeDtypeStruct((E, 2 * H), jnp.float32),
        ],
    )(xa, xb, cd, ea, wr, wea_t, we2_t, be2r, wc1_t, bc1r, wc2r)


NPT = 320
CH2 = 2000
NG2 = CH2 // 16
SCAP = 2048
LW = E + SCAP


def _scan_body(row_hbm, plist_hbm, cnt_hbm,
               rows0, rows1, est, cntbuf, sem0, sem1):
    c = lax.axis_index("c")
    s = lax.axis_index("s")
    w = c * NS + s
    lo = w * NPT
    iota = lax.iota(jnp.int32, 16)
    base = w * LW

    def scan_rows(rows, carry):
        def group_body(g, carry):
            cnt, idsp = carry
            v = rows[pl.ds(g * 16, 16)]
            d = v - lo
            m = d.astype(jnp.uint32) < jnp.uint32(NPT)
            plsc.store_compressed(est.at[pl.ds(cnt, 16)], idsp + d, mask=m)
            return (cnt + jnp.sum(m.astype(jnp.int32)), idsp + 16 * 512)

        cnt, nf, idsp = carry
        cnt, idsp = lax.fori_loop(0, NG2, group_body, (cnt, idsp))

        @pl.when(cnt >= SCAP)
        def _():
            pltpu.sync_copy(est.at[pl.ds(0, SCAP)],
                            plist_hbm.at[pl.ds(base + nf * SCAP, SCAP)])
            for k in range(CH2 // 16 + 1):
                est[pl.ds(k * 16, 16)] = est[pl.ds(SCAP + k * 16, 16)]

        hit = (cnt >= SCAP).astype(jnp.int32)
        return (cnt - hit * SCAP, nf + hit, idsp)

    nd = E // CH2 // 2

    def chunk_pair(t, carry):
        pltpu.make_async_copy(row_hbm.at[pl.ds(0, CH2)], rows0, sem0).wait()
        pltpu.async_copy(row_hbm.at[pl.ds((2 * t + 1) * CH2, CH2)],
                         rows1, sem1)
        carry = scan_rows(rows0, carry)

        @pl.when(t + 1 < nd)
        def _():
            pltpu.async_copy(row_hbm.at[pl.ds((2 * t + 2) * CH2, CH2)],
                             rows0, sem0)

        pltpu.make_async_copy(row_hbm.at[pl.ds(0, CH2)], rows1, sem1).wait()
        carry = scan_rows(rows1, carry)
        return carry

    pltpu.async_copy(row_hbm.at[pl.ds(0, CH2)], rows0, sem0)
    cnt, nf, _ = lax.fori_loop(
        0, nd, chunk_pair,
        (jnp.full((), 0, jnp.int32), jnp.full((), 0, jnp.int32),
         iota * 512))

    pltpu.sync_copy(est.at[pl.ds(0, SCAP)],
                    plist_hbm.at[pl.ds(base + nf * SCAP, SCAP)])
    cntbuf[pl.ds(0, 16)] = jnp.broadcast_to(nf * SCAP + cnt, (16,))
    pltpu.sync_copy(cntbuf, cnt_hbm.at[pl.ds(w * 16, 16)])


def _scan_call(row):
    mesh = plsc.VectorSubcoreMesh(core_axis_name="c", subcore_axis_name="s",
                                  num_cores=NC, num_subcores=NS)
    f = pl.kernel(
        _scan_body,
        out_type=(
            jax.ShapeDtypeStruct((NW * LW,), jnp.int32),
            jax.ShapeDtypeStruct((NW * 16,), jnp.int32),
        ),
        mesh=mesh,
        compiler_params=pltpu.CompilerParams(needs_layout_passes=False),
        scratch_types=(
            pltpu.VMEM((CH2,), jnp.int32),
            pltpu.VMEM((CH2,), jnp.int32),
            pltpu.VMEM((SCAP + CH2 + 32,), jnp.int32),
            pltpu.VMEM((16,), jnp.int32),
            pltpu.SemaphoreType.DMA,
            pltpu.SemaphoreType.DMA,
        ),
    )
    return f(row)


def _accum_body(eftr_hbm, plist_hbm, cnt_hbm, zf_hbm, zc_hbm,
                ag_hbm, ac_hbm,
                land0, land1, seb0, srb0, seb1, srb1, gbuf0, gbuf1,
                acc16, accf, cbuf, semi0, semi1, semg0, semg1):
    c = lax.axis_index("c")
    s = lax.axis_index("s")
    w = c * NS + s
    iota = lax.iota(jnp.int32, 16)
    base = w * LW

    pltpu.sync_copy(zf_hbm, accf)
    pltpu.sync_copy(zc_hbm, acc16)
    pltpu.sync_copy(cnt_hbm.at[pl.ds(w * 16, 16)], cbuf)
    cnt = jnp.sum(jnp.where(iota == 0, cbuf[pl.ds(0, 16)], 0))
    nb = (cnt + CH - 1) // CH

    sets = ((land0, seb0, srb0, gbuf0, semi0, semg0),
            (land1, seb1, srb1, gbuf1, semi1, semg1))

    def fire_idx(j, b):
        land, semi = sets[b][0], sets[b][4]
        pltpu.async_copy(plist_hbm.at[pl.ds(base + j * CH, CH)], land, semi)

    def wait_idx(b):
        land, semi = sets[b][0], sets[b][4]
        pltpu.make_async_copy(plist_hbm.at[pl.ds(0, CH)], land, semi).wait()

    def snapshot(j, b):
        land, seb, srb = sets[b][0], sets[b][1], sets[b][2]
        for k in range(CH // 16):
            pv = land[pl.ds(k * 16, 16)]
            sel = (j * CH + k * 16 + iota) < cnt
            eb = pv // 512
            rb = pv - eb * 512
            seb[pl.ds(k * 16, 16)] = jnp.where(sel, eb, 0)
            srb[pl.ds(k * 16, 16)] = jnp.where(
                sel, rb, jnp.full((16,), NPT, jnp.int32))

    def fire_gather(b):
        seb, gbuf, semg = sets[b][1], sets[b][3], sets[b][5]
        pltpu.async_copy(eftr_hbm.at[seb], gbuf, semg)

    def wait_gather(b):
        seb, gbuf, semg = sets[b][1], sets[b][3], sets[b][5]
        pltpu.make_async_copy(eftr_hbm.at[seb], gbuf, semg).wait()

    def accum(b):
        srb, gbuf = sets[b][2], sets[b][3]
        for j in range(CH):
            k, l = divmod(j, 16)
            rv = srb[pl.ds(k * 16, 16)]
            rj = rv.at[jnp.full((16,), l, jnp.int32)].get(
                mode="promise_in_bounds")
            mask = rj < NPT
            plsc.addupdate_scatter(acc16, [rj * 16 + iota],
                                   gbuf[j, pl.ds(H, 16)], mask=mask)
            rbase = rj * H + iota
            for g in range(H // 16):
                plsc.addupdate_scatter(accf, [rbase + g * 16],
                                       gbuf[j, pl.ds(g * 16, 16)], mask=mask)

    def step(j, b):
        wait_gather(b)

        @pl.when(j + 1 < nb)
        def _():
            wait_idx(1 - b)
            snapshot(j + 1, 1 - b)
            fire_gather(1 - b)

            @pl.when(j + 2 < nb)
            def _():
                fire_idx(j + 2, b)

        accum(b)

    @pl.when(nb > 0)
    def _():
        fire_idx(0, 0)
        wait_idx(0)
        snapshot(jnp.full((), 0, jnp.int32), 0)
        fire_gather(0)

        @pl.when(nb > 1)
        def _():
            fire_idx(1, 1)

    def pair(t, carry):
        j0 = 2 * t

        @pl.when(j0 < nb)
        def _():
            step(j0, 0)

        @pl.when(j0 + 1 < nb)
        def _():
            step(j0 + 1, 1)

        return carry

    lax.fori_loop(0, (nb + 1) // 2, pair, jnp.full((), 0, jnp.int32))

    pltpu.sync_copy(accf, ag_hbm.at[pl.ds(w * NPT * H, NPT * H)])
    pltpu.sync_copy(acc16, ac_hbm.at[pl.ds(w * NPT * 16, NPT * 16)])


def _accum_call(eftr, plist, pcnt, zf, zc):
    mesh = plsc.VectorSubcoreMesh(core_axis_name="c", subcore_axis_name="s",
                                  num_cores=NC, num_subcores=NS)
    f = pl.kernel(
        _accum_body,
        out_type=(
            jax.ShapeDtypeStruct((NW * NPT * H,), jnp.float32),
            jax.ShapeDtypeStruct((NW * NPT * 16,), jnp.float32),
        ),
        mesh=mesh,
        compiler_params=pltpu.CompilerParams(needs_layout_passes=False),
        scratch_types=(
            pltpu.VMEM((CH,), jnp.int32),
            pltpu.VMEM((CH,), jnp.int32),
            pltpu.VMEM((CH,), jnp.int32),
            pltpu.VMEM((CH,), jnp.int32),
            pltpu.VMEM((CH,), jnp.int32),
            pltpu.VMEM((CH,), jnp.int32),
            pltpu.VMEM((CH, 2 * H), jnp.float32),
            pltpu.VMEM((CH, 2 * H), jnp.float32),
            pltpu.VMEM((NPT * 16,), jnp.float32),
            pltpu.VMEM((NPT * H,), jnp.float32),
            pltpu.VMEM((16,), jnp.int32),
            pltpu.SemaphoreType.DMA,
            pltpu.SemaphoreType.DMA,
            pltpu.SemaphoreType.DMA,
            pltpu.SemaphoreType.DMA,
        ),
    )
    return f(eftr, plist, pcnt, zf, zc)


def _node_body(h_ref, ag_ref, ac_ref, cp_ref,
               wn1a_ref, wn1b_ref, bn1_ref, wn2_ref, bn2_ref,
               ho_ref, co_ref):
    h = h_ref[...]
    aggh = ag_ref[...]
    aggc = ac_ref[...]
    nm = _silu(jnp.dot(h, wn1a_ref[...], preferred_element_type=jnp.float32)
               + jnp.dot(aggh, wn1b_ref[...], preferred_element_type=jnp.float32)
               + bn1_ref[...])
    ho_ref[...] = h + jnp.dot(nm, wn2_ref[...], preferred_element_type=jnp.float32) + bn2_ref[...]
    lane = lax.broadcasted_iota(jnp.int32, (BN, 16), 1)
    cnt = jnp.sum(jnp.where(lane == 3, aggc, 0.0), axis=1, keepdims=True)
    co_ref[...] = cp_ref[...] + aggc[:, :4] / jnp.maximum(cnt, 1.0)


def _node_call(h, ag, ac, cp4, wn1a_t, wn1b_t, bn1r, wn2_t, bn2r):
    grid = (N // BN,)
    return pl.pallas_call(
        _node_body,
        grid=grid,
        in_specs=[
            pl.BlockSpec((BN, D), lambda i: (i, 0)),
            pl.BlockSpec((BN, H), lambda i: (i, 0)),
            pl.BlockSpec((BN, 16), lambda i: (i, 0)),
            pl.BlockSpec((BN, 4), lambda i: (i, 0)),
            pl.BlockSpec((D, H), lambda i: (0, 0)),
            pl.BlockSpec((H, H), lambda i: (0, 0)),
            pl.BlockSpec((1, H), lambda i: (0, 0)),
            pl.BlockSpec((H, D), lambda i: (0, 0)),
            pl.BlockSpec((1, D), lambda i: (0, 0)),
        ],
        out_specs=[
            pl.BlockSpec((BN, D), lambda i: (i, 0)),
            pl.BlockSpec((BN, 4), lambda i: (i, 0)),
        ],
        out_shape=[
            jax.ShapeDtypeStruct((N, D), jnp.float32),
            jax.ShapeDtypeStruct((N, 4), jnp.float32),
        ],
    )(h, ag, ac, cp4, wn1a_t, wn1b_t, bn1r, wn2_t, bn2r)


def kernel(h, edge_index, coord, edge_attr, We1, be1, We2, be2,
           Wn1, bn1, Wn2, bn2, Wc1, bc1, Wc2):
    row = edge_index[0].astype(jnp.int32)
    col = edge_index[1].astype(jnp.int32)
    cp4 = jnp.zeros((N, 4), jnp.float32).at[:, :3].set(coord)

    wa_t = We1[:, :D].T
    wb_t = We1[:, D:2 * D].T
    wr = We1[:, 2 * D].reshape(1, H)
    wea_t = We1[:, 2 * D + 1:].T
    we2_t = We2.T
    wc1_t = Wc1.T
    wc2r = Wc2.reshape(1, H)
    wn1a_t = Wn1[:, :D].T
    wn1b_t = Wn1[:, D:].T
    wn2_t = Wn2.T

    ta, tb = _pre_call(h, wa_t, wb_t, be1.reshape(1, H))
    xa, xb, cd = _gather_call(ta, tb, row, col, cp4.reshape(-1))
    plist, pcnt = _scan_call(row)
    (eftr,) = _edge_call(xa, xb, cd, edge_attr, wr, wea_t, we2_t,
                      be2.reshape(1, H), wc1_t, bc1.reshape(1, H), wc2r)
    zf = jnp.zeros((NPT * H,), jnp.float32)
    zc = jnp.zeros((NPT * 16,), jnp.float32)
    ag, ac = _accum_call(eftr, plist, pcnt, zf, zc)
    ag = ag.reshape(NW * NPT, H)
    ac = ac.reshape(NW * NPT, 16)
    ho, co = _node_call(h, ag, ac, cp4, wn1a_t, wn1b_t,
                        bn1.reshape(1, H), wn2_t, bn2.reshape(1, D))
    return (ho, co[:, :3], edge_attr)
